# baseline (speedup 1.0000x reference)
import functools

import jax
import jax.numpy as jnp
from jax import lax
from jax.experimental import pallas as pl
from jax.experimental.pallas import tpu as pltpu
from jax.experimental.pallas import tpu_sc as plsc

_K = 2048
_ROWS = 128
_COLS = 32768
_NW = 32
_RPW = _ROWS // _NW
_NVEC = _COLS // 16


def _scalar(v):
    return jnp.max(v)


def _sc_body(x_hbm, out_hbm, xb, ca, cb, hist, totals, tb):
    cid = lax.axis_index("c")
    sid = lax.axis_index("s")
    wid = cid * 16 + sid
    lane = lax.iota(jnp.int32, 16)
    ones = jnp.ones((16,), jnp.int32)
    zeros16 = jnp.zeros((16,), jnp.int32)

    def clear_hist():
        def body(i, _):
            hist[pl.ds(i * 16, 16)] = zeros16
            return 0
        lax.fori_loop(0, 256, body, 0)

    def reduce_hist():
        def body(g, _):
            acc = zeros16
            for l in range(16):
                acc = acc + hist[pl.ds(l * 256 + g * 16, 16)]
            totals[pl.ds(g * 16, 16)] = acc
            return 0
        lax.fori_loop(0, 16, body, 0)

    def find_bucket(r):
        def body(gi, carry):
            S, found, bst, above, cnt = carry
            g = 15 - gi
            t = totals[pl.ds(g * 16, 16)]
            rv = lax.rev(t, (0,))
            cs = plsc.cumsum(rv)
            tot = cs + S
            crossed = tot >= r
            pcs = _scalar(plsc.all_reduce_population_count(crossed))
            has = pcs > 0
            pos = _scalar(plsc.all_reduce_ffs(crossed))
            cs_at = _scalar(jnp.where(lane == pos, tot, 0))
            cnt_at = _scalar(jnp.where(lane == pos, rv, 0))
            bucket = g * 16 + 15 - pos
            newfound = has & (found == 0)
            bst = jnp.where(newfound, bucket, bst)
            above = jnp.where(newfound, cs_at - cnt_at, above)
            cnt = jnp.where(newfound, cnt_at, cnt)
            found = jnp.where(has, jnp.int32(1), found)
            S = _scalar(tot)
            return S, found, bst, above, cnt
        z = jnp.int32(0)
        _, _, bst, above, cnt = lax.fori_loop(0, 16, body, (z, z, z, z, z))
        return bst, above, cnt

    def hist_round(src, csrc, sh, first):
        clear_hist()

        def body(i, _):
            k = src[pl.ds(i * 16, 16)]
            if first:
                b = (k >> 24) + 128
                plsc.addupdate_scatter(hist, [lane * 256 + b], ones)
            else:
                valid = (i * 16 + lane) < csrc
                b = (k >> sh) & 0xFF
                plsc.addupdate_scatter(hist, [lane * 256 + b], ones, mask=valid)
            return 0
        nv = _NVEC if first else (csrc + 15) // 16
        lax.fori_loop(0, nv, body, 0)
        reduce_hist()

    def compact(src, dst, csrc, sh, first, bst):
        def body(i, off):
            k = src[pl.ds(i * 16, 16)]
            if first:
                m = ((k >> 24) + 128) == bst
            else:
                m = (((k >> sh) & 0xFF) == bst) & ((i * 16 + lane) < csrc)
            plsc.store_compressed(dst.at[pl.ds(off, 16)], k, mask=m)
            return off + _scalar(plsc.all_reduce_population_count(m))
        nv = _NVEC if first else (csrc + 15) // 16
        return lax.fori_loop(0, nv, body, jnp.int32(0))

    def row_body(j, _):
        row = wid * _RPW + j
        pltpu.sync_copy(x_hbm.at[row], xb)

        clear_hist()

        def scan1(i, _):
            v = xb[pl.ds(i * 16, 16)]
            sb = plsc.bitcast(v, jnp.int32)
            key = jnp.where(sb < 0, sb ^ jnp.int32(0x7FFFFFFF), sb)
            ca[pl.ds(i * 16, 16)] = key
            b = (key >> 24) + 128
            plsc.addupdate_scatter(hist, [lane * 256 + b], ones)
            return 0
        lax.fori_loop(0, _NVEC, scan1, 0)
        reduce_hist()

        r = jnp.int32(_K)
        b1, above, c1 = find_bucket(r)
        r = r - above
        c1 = compact(ca, cb, jnp.int32(_COLS), 24, True, b1)

        hist_round(cb, c1, 16, False)
        b2, above, c2 = find_bucket(r)
        r = r - above
        c2 = compact(cb, ca, c1, 16, False, b2)

        hist_round(ca, c2, 8, False)
        b3, above, c3 = find_bucket(r)
        r = r - above
        c3 = compact(ca, cb, c2, 8, False, b3)

        hist_round(cb, c3, 0, False)
        b4, _, _ = find_bucket(r)

        T = (((((b1 - 128) << 8) | b2) << 8 | b3) << 8) | b4
        tvec = T + jnp.zeros((16,), jnp.int32)
        tfv = plsc.bitcast(
            jnp.where(tvec < 0, tvec ^ jnp.int32(0x7FFFFFFF), tvec), jnp.float32)
        tb[...] = jnp.where(lane == j, tfv, tb[...])
        return 0

    lax.fori_loop(0, _RPW, row_body, 0)
    pltpu.sync_copy(tb, out_hbm.at[wid])


def _sc_thresholds(x):
    mesh = plsc.VectorSubcoreMesh(
        core_axis_name="c", subcore_axis_name="s", num_cores=2, num_subcores=16)
    f = pl.kernel(
        _sc_body,
        out_type=jax.ShapeDtypeStruct((_NW, 16), jnp.float32),
        mesh=mesh,
        scratch_types=[
            pltpu.VMEM((_COLS,), jnp.float32),
            pltpu.VMEM((_COLS + 16,), jnp.int32),
            pltpu.VMEM((_COLS + 16,), jnp.int32),
            pltpu.VMEM((4096,), jnp.int32),
            pltpu.VMEM((256,), jnp.int32),
            pltpu.VMEM((16,), jnp.float32),
        ],
        compiler_params=pltpu.CompilerParams(needs_layout_passes=False),
    )
    return f(x)


def _mask_body(x_ref, t_ref, o_ref):
    xv = x_ref[...]
    t = t_ref[...]
    o_ref[...] = jnp.where(xv >= t, jnp.maximum(xv, 0.0), 0.0)


@jax.jit
def kernel(x):
    th = _sc_thresholds(x)
    th = th[:, :_RPW].reshape(_ROWS, 1)
    return pl.pallas_call(
        _mask_body,
        grid=(8,),
        in_specs=[
            pl.BlockSpec((16, _COLS), lambda i: (i, 0)),
            pl.BlockSpec((16, 1), lambda i: (i, 0)),
        ],
        out_specs=pl.BlockSpec((16, _COLS), lambda i: (i, 0)),
        out_shape=jax.ShapeDtypeStruct((_ROWS, _COLS), jnp.float32),
    )(x, th)
